# Initial kernel scaffold; baseline (speedup 1.0000x reference)
import jax
import jax.numpy as jnp
from jax.experimental import pallas as pl


def kernel(route_emb, route_len, adj_matrices, W, att_src, att_dst, bias, ln_gamma, ln_beta):
    raise NotImplementedError("write your pallas kernel here")


# speedup vs baseline: 54.9329x; 54.9329x over previous
import functools

import jax
import jax.numpy as jnp
from jax import lax
from jax.experimental import pallas as pl
from jax.experimental.pallas import tpu as pltpu
from jax.experimental.pallas import tpu_sc as plsc

B = 16
L = 4096
D = 128
H = 4
DH = 32
E_PER = 16384
NMAX = B * L
W_WIN = 1024
NEG_SLOPE = 0.2
LN_EPS = 1e-5
NSLOT = B * W_WIN
SLOTS_PER_WORKER = NSLOT // 32
NUM_ROWS = D + 2 * H


def _sc_mesh():
    return plsc.VectorSubcoreMesh(core_axis_name="c", subcore_axis_name="s",
                                  num_cores=2, num_subcores=16)


def _sc1_body(aux_hbm, re2d_hbm, out_hbm, aux_v, idx_buf, rows_buf, sem):
    c = lax.axis_index("c")
    s = lax.axis_index("s")
    b = s
    base = b * W_WIN + c * SLOTS_PER_WORKER
    pltpu.sync_copy(aux_hbm, aux_v)
    lanes = lax.iota(jnp.int32, 16)
    nv = aux_v[pl.ds(15 * 16, 16)]
    nm1 = jnp.maximum(nv - 1, 0)
    offs_b = plsc.load_gather(aux_v, [256 + b * 16 + lanes])

    def grp(t, carry):
        g = offs_b + c * SLOTS_PER_WORKER + t * 16 + lanes
        g = jnp.minimum(g, nm1)
        bog = jnp.zeros((16,), jnp.int32)
        for bp in range(B):
            cum_bp = aux_v[pl.ds(bp * 16, 16)]
            bog = bog + jnp.where(g >= cum_bp, 1, 0).astype(jnp.int32)
        bog = jnp.minimum(bog, B - 1)
        obog = plsc.load_gather(aux_v, [768 + bog])
        row = bog * L + (g - obog)
        row = jnp.minimum(jnp.maximum(row, 0), NMAX - 1)
        idx_buf[pl.ds(t * 16, 16)] = row
        return carry

    lax.fori_loop(0, SLOTS_PER_WORKER // 16, grp, 0)
    copies = []
    for q in range(SLOTS_PER_WORKER // 128):
        copies.append(pltpu.async_copy(
            re2d_hbm.at[idx_buf.at[pl.ds(q * 128, 128)]],
            rows_buf.at[pl.ds(q * 128, 128)], sem))
    for cp in copies:
        cp.wait()
    pltpu.sync_copy(rows_buf, out_hbm.at[pl.ds(base, SLOTS_PER_WORKER), :])


def _sc1(aux, re2d):
    kern = pl.kernel(
        _sc1_body,
        out_type=jax.ShapeDtypeStruct((NSLOT, D), jnp.float32),
        mesh=_sc_mesh(),
        compiler_params=pltpu.CompilerParams(needs_layout_passes=False),
        scratch_types=[
            pltpu.VMEM((784,), jnp.int32),
            pltpu.VMEM((SLOTS_PER_WORKER,), jnp.int32),
            pltpu.VMEM((SLOTS_PER_WORKER, D), jnp.float32),
            pltpu.SemaphoreType.DMA,
        ],
    )
    return kern(aux, re2d)


def _tc1_kernel(h_ref, w_ref, asad_mat_ref, xpt_ref, asad_ref):
    h = h_ref[...]
    w = w_ref[...]
    xpt = lax.dot_general(w, h, (((0,), (1,)), ((), ())),
                          preferred_element_type=jnp.float32)
    xpt_ref[...] = xpt
    asad_ref[...] = jnp.dot(asad_mat_ref[...], xpt,
                            preferred_element_type=jnp.float32)


def _tc1(h_slot, w, asad_mat):
    chunk = 2048
    grid = (NSLOT // chunk,)
    return pl.pallas_call(
        _tc1_kernel,
        grid=grid,
        in_specs=[
            pl.BlockSpec((chunk, D), lambda i: (i, 0)),
            pl.BlockSpec((D, D), lambda i: (0, 0)),
            pl.BlockSpec((8, D), lambda i: (0, 0)),
        ],
        out_specs=[
            pl.BlockSpec((D, chunk), lambda i: (0, i)),
            pl.BlockSpec((8, chunk), lambda i: (0, i)),
        ],
        out_shape=[
            jax.ShapeDtypeStruct((D, NSLOT), jnp.float32),
            jax.ShapeDtypeStruct((8, NSLOT), jnp.float32),
        ],
    )(h_slot, w, asad_mat)


def _sc2_body(aux_hbm, adj_hbm, xpt_hbm, asad_hbm, out_hbm, aux_v,
              adj0_buf, adj1_buf, as_buf, ad_buf, xpt_buf, num_buf, den_buf,
              sem):
    c = lax.axis_index("c")
    s = lax.axis_index("s")
    b = s
    pltpu.sync_copy(aux_hbm, aux_v)
    lanes = lax.iota(jnp.int32, 16)
    smax = plsc.load_gather(aux_v, [512 + b * 16 + lanes])

    pltpu.sync_copy(adj_hbm.at[b, 0], adj0_buf)
    pltpu.sync_copy(adj_hbm.at[b, 1], adj1_buf)

    def zero_vec(ref, nwords):
        def zbody(i, carry):
            ref[pl.ds(i * 16, 16)] = jnp.zeros((16,), jnp.float32)
            return carry
        lax.fori_loop(0, nwords // 16, zbody, 0)

    for hh in range(2):
        h = 2 * c + hh
        f0 = DH * h
        zero_vec(den_buf, W_WIN)
        zero_vec(num_buf, DH * W_WIN)
        pads = []
        for fi in range(DH):
            pads.append(pltpu.async_copy(
                den_buf, out_hbm.at[f0 + fi, b, pl.ds(W_WIN, W_WIN)], sem))
        pads.append(pltpu.async_copy(
            den_buf, out_hbm.at[D + h, b, pl.ds(W_WIN, W_WIN)], sem))
        pads.append(pltpu.async_copy(
            den_buf, out_hbm.at[D + H + h, b, pl.ds(0, W_WIN)], sem))
        pads.append(pltpu.async_copy(
            den_buf, out_hbm.at[D + H + h, b, pl.ds(W_WIN, W_WIN)], sem))
        for cp in pads:
            cp.wait()
        pltpu.sync_copy(asad_hbm.at[h, pl.ds(b * W_WIN, W_WIN)], as_buf)
        pltpu.sync_copy(asad_hbm.at[H + h, pl.ds(b * W_WIN, W_WIN)], ad_buf)
        stages = []
        for fi in range(DH):
            stages.append(pltpu.async_copy(
                xpt_hbm.at[f0 + fi, pl.ds(b * W_WIN, W_WIN)],
                xpt_buf.at[pl.ds(fi * W_WIN, W_WIN)], sem))
        for cp in stages:
            cp.wait()

        def grp(t, carry):
            a0 = adj0_buf[pl.ds(t * 16, 16)]
            a1 = adj1_buf[pl.ds(t * 16, 16)]
            sl = jnp.minimum(a0, smax)
            asv = plsc.load_gather(as_buf, [sl])
            adv = plsc.load_gather(ad_buf, [a1])
            al = asv + adv
            al = jnp.maximum(al, al * NEG_SLOPE)
            ex = jnp.exp(al)
            plsc.addupdate_scatter(den_buf, [a1], ex)
            for fi in range(DH):
                off = fi * W_WIN
                x = plsc.load_gather(xpt_buf, [sl + off])
                plsc.addupdate_scatter(num_buf, [a1 + off], x * ex)
            return carry

        lax.fori_loop(0, E_PER // 16, grp, 0)

        outs = []
        for fi in range(DH):
            outs.append(pltpu.async_copy(
                num_buf.at[pl.ds(fi * W_WIN, W_WIN)],
                out_hbm.at[f0 + fi, b, pl.ds(0, W_WIN)], sem))
        outs.append(pltpu.async_copy(
            den_buf, out_hbm.at[D + h, b, pl.ds(0, W_WIN)], sem))
        for cp in outs:
            cp.wait()


def _sc2(aux, adj, xpt, asad):
    kern = pl.kernel(
        _sc2_body,
        out_type=jax.ShapeDtypeStruct((NUM_ROWS, B, 2 * W_WIN), jnp.float32),
        mesh=_sc_mesh(),
        compiler_params=pltpu.CompilerParams(needs_layout_passes=False),
        scratch_types=[
            pltpu.VMEM((784,), jnp.int32),
            pltpu.VMEM((E_PER,), jnp.int32),
            pltpu.VMEM((E_PER,), jnp.int32),
            pltpu.VMEM((W_WIN,), jnp.float32),
            pltpu.VMEM((W_WIN,), jnp.float32),
            pltpu.VMEM((DH * W_WIN,), jnp.float32),
            pltpu.VMEM((DH * W_WIN,), jnp.float32),
            pltpu.VMEM((W_WIN,), jnp.float32),
            pltpu.SemaphoreType.DMA,
        ],
    )
    return kern(aux, adj, xpt, asad)


def _tc3_kernel(numtp_ref, re_ref, ol_ref, bias_ref, gamma_ref, beta_ref,
                out_ref, trn_ref, trd_ref):
    bq = pl.program_id(0)
    ic = pl.program_id(1)
    bias = bias_ref[...]
    len_b = ol_ref[B + bq]

    def finish(y, row0):
        mu = jnp.mean(y, axis=1, keepdims=True)
        var = jnp.mean(jnp.square(y - mu), axis=1, keepdims=True)
        ln = ((y - mu) * lax.rsqrt(var + LN_EPS) * gamma_ref[...]
              + beta_ref[...])
        row = row0 + lax.broadcasted_iota(jnp.int32, (W_WIN, 1), 0)
        out_ref[0] = jnp.where(row < len_b, ln, 0.0)

    @pl.when(ic == 0)
    def _():
        blk_n = numtp_ref[:, pl.ds(bq, 1), :][:D]
        blk_d = numtp_ref[:, pl.ds(bq, 1), :][D:]
        trn_ref[bq] = jnp.transpose(blk_n.reshape(D, 2 * W_WIN))
        trd_ref[bq] = jnp.transpose(blk_d.reshape(2 * H, 2 * W_WIN))

        offs_b = ol_ref[bq]

        def body(bp, carry):
            acc_n, acc_d = carry
            delta = offs_b - ol_ref[bp]
            ok = jnp.logical_and(bp <= bq, delta < W_WIN)

            def yes(cr):
                an, ad = cr
                an = an + trn_ref[bp, pl.ds(delta, W_WIN), :]
                ad = ad + trd_ref[bp, pl.ds(delta, W_WIN), :]
                return an, ad

            return lax.cond(ok, yes, lambda cr: cr, (acc_n, acc_d))

        acc_n, acc_d = lax.fori_loop(
            0, B, body,
            (jnp.zeros((W_WIN, D), jnp.float32),
             jnp.zeros((W_WIN, 2 * H), jnp.float32)))

        den_exp = jnp.concatenate(
            [jnp.broadcast_to(acc_d[:, h:h + 1], (W_WIN, DH))
             for h in range(H)], axis=1)
        msg = acc_n / (den_exp + 1e-16)
        finish(re_ref[0] + bias + msg, 0)

    @pl.when(ic != 0)
    def _():
        finish(re_ref[0] + bias, ic * W_WIN)


def _tc3(numtp, route_emb, offslens, bias, gamma, beta):
    return pl.pallas_call(
        _tc3_kernel,
        grid=(B, L // W_WIN),
        in_specs=[
            pl.BlockSpec((NUM_ROWS, B, 2 * W_WIN), lambda b, i: (0, 0, 0)),
            pl.BlockSpec((1, W_WIN, D), lambda b, i: (b, i, 0)),
            pl.BlockSpec(memory_space=pltpu.SMEM),
            pl.BlockSpec((1, D), lambda b, i: (0, 0)),
            pl.BlockSpec((1, D), lambda b, i: (0, 0)),
            pl.BlockSpec((1, D), lambda b, i: (0, 0)),
        ],
        out_specs=pl.BlockSpec((1, W_WIN, D), lambda b, i: (b, i, 0)),
        out_shape=jax.ShapeDtypeStruct((B, L, D), jnp.float32),
        scratch_shapes=[
            pltpu.VMEM((B, 2 * W_WIN, D), jnp.float32),
            pltpu.VMEM((B, 2 * W_WIN, 2 * H), jnp.float32),
        ],
    )(numtp, route_emb, offslens, bias, gamma, beta)


def kernel(route_emb, route_len, adj_matrices, W, att_src, att_dst, bias,
           ln_gamma, ln_beta):
    route_len = route_len.astype(jnp.int32)
    cum = jnp.cumsum(route_len)
    offsets = cum - route_len
    n_total = cum[B - 1]
    smax = jnp.maximum(n_total - 1 - offsets, 0)
    aux = jnp.concatenate([
        jnp.broadcast_to(cum[:, None], (B, 16)).reshape(-1),
        jnp.broadcast_to(offsets[:, None], (B, 16)).reshape(-1),
        jnp.broadcast_to(smax[:, None], (B, 16)).reshape(-1),
        offsets,
    ]).astype(jnp.int32)
    offslens = jnp.concatenate([offsets, route_len]).astype(jnp.int32)

    re2d = route_emb.reshape(NMAX, D)
    eye = jnp.eye(H, dtype=jnp.float32)
    m_src = jnp.einsum("hk,hd->khd", eye, att_src).reshape(H, D)
    m_dst = jnp.einsum("hk,hd->khd", eye, att_dst).reshape(H, D)
    asad_mat = jnp.concatenate([m_src, m_dst], axis=0)

    h_slot = _sc1(aux, re2d)
    xpt, asad = _tc1(h_slot, W, asad_mat)
    numtp = _sc2(aux, adj_matrices.astype(jnp.int32), xpt, asad)
    out = _tc3(numtp, route_emb, offslens, bias.reshape(1, D),
               ln_gamma.reshape(1, D), ln_beta.reshape(1, D))
    return out
